# baseline (speedup 1.0000x reference)
import jax
import jax.numpy as jnp
from jax.experimental import pallas as pl
from jax.experimental.pallas import tpu as pltpu


_TILE = 1024


def _normalize_body(cb_ref, cbn_ref):
    cb = cb_ref[...]
    norm = jnp.sqrt(jnp.sum(cb * cb, axis=1, keepdims=True))
    cbn_ref[...] = cb / (norm + 1e-12)


def _vq_body(z_ref, cbn_ref, out_ref):
    cbn = cbn_ref[...]
    zb = z_ref[...]
    znorm = jnp.sqrt(jnp.sum(zb * zb, axis=1, keepdims=True))
    zn = zb / (znorm + 1e-12)
    dist = jax.lax.dot_general(
        zn, cbn, dimension_numbers=(((1,), (1,)), ((), ())),
        preferred_element_type=jnp.float32)
    ind = jnp.argmax(dist, axis=1)
    iota = jax.lax.broadcasted_iota(jnp.int32, dist.shape, 1)
    onehot = (iota == ind[:, None]).astype(jnp.float32)
    out_ref[...] = jnp.dot(onehot, cbn, preferred_element_type=jnp.float32)


def kernel(z, codebook):
    shape = z.shape
    d = shape[-1]
    flat = z.reshape(-1, d)
    n = flat.shape[0]
    cbn = pl.pallas_call(
        _normalize_body,
        out_shape=jax.ShapeDtypeStruct(codebook.shape, jnp.float32),
    )(codebook)
    out = pl.pallas_call(
        _vq_body,
        grid=(n // _TILE,),
        in_specs=[
            pl.BlockSpec((_TILE, d), lambda i: (i, 0)),
            pl.BlockSpec(codebook.shape, lambda i: (0, 0)),
        ],
        out_specs=pl.BlockSpec((_TILE, d), lambda i: (i, 0)),
        out_shape=jax.ShapeDtypeStruct((n, d), jnp.float32),
        compiler_params=pltpu.CompilerParams(
            dimension_semantics=("parallel",)),
    )(flat, cbn)
    return out.reshape(shape)


# speedup vs baseline: 1.0271x; 1.0271x over previous
import jax
import jax.numpy as jnp
from jax.experimental import pallas as pl
from jax.experimental.pallas import tpu as pltpu


_TILE = 2048


def _vq_body(z_ref, cb_ref, out_ref):
    cb = cb_ref[...]
    norm = jnp.sqrt(jnp.sum(cb * cb, axis=1, keepdims=True))
    cbn = cb / (norm + 1e-12)
    zb = z_ref[...]
    znorm = jnp.sqrt(jnp.sum(zb * zb, axis=1, keepdims=True))
    zn = zb / (znorm + 1e-12)
    dist = jax.lax.dot_general(
        zn, cbn, dimension_numbers=(((1,), (1,)), ((), ())),
        preferred_element_type=jnp.float32)
    ind = jnp.argmax(dist, axis=1)
    iota = jax.lax.broadcasted_iota(jnp.int32, dist.shape, 1)
    onehot = (iota == ind[:, None]).astype(jnp.float32)
    out_ref[...] = jnp.dot(onehot, cbn, preferred_element_type=jnp.float32)


def kernel(z, codebook):
    shape = z.shape
    d = shape[-1]
    flat = z.reshape(-1, d)
    n = flat.shape[0]
    out = pl.pallas_call(
        _vq_body,
        grid=(n // _TILE,),
        in_specs=[
            pl.BlockSpec((_TILE, d), lambda i: (i, 0)),
            pl.BlockSpec(codebook.shape, lambda i: (0, 0)),
        ],
        out_specs=pl.BlockSpec((_TILE, d), lambda i: (i, 0)),
        out_shape=jax.ShapeDtypeStruct((n, d), jnp.float32),
        compiler_params=pltpu.CompilerParams(
            dimension_semantics=("parallel",)),
    )(flat, codebook)
    return out.reshape(shape)


# speedup vs baseline: 1.2355x; 1.2029x over previous
import jax
import jax.numpy as jnp
from jax.experimental import pallas as pl
from jax.experimental.pallas import tpu as pltpu


_TILE = 2048


def _vq_body(z_ref, cb_ref, out_ref):
    cb = cb_ref[...]
    k = cb.shape[0]
    norm = jnp.sqrt(jnp.sum(cb * cb, axis=1, keepdims=True))
    cbn = cb / (norm + 1e-12)
    zb = z_ref[...]
    znorm = jnp.sqrt(jnp.sum(zb * zb, axis=1, keepdims=True))
    zn = zb / (znorm + 1e-12)
    dist = jax.lax.dot_general(
        zn, cbn, dimension_numbers=(((1,), (1,)), ((), ())),
        preferred_element_type=jnp.float32)
    m = jnp.max(dist, axis=1, keepdims=True)
    hot = (dist == m).astype(jnp.float32)
    aug = jnp.concatenate([cbn, jnp.ones((k, 1), jnp.float32)], axis=1)
    res = jnp.dot(hot, aug, preferred_element_type=jnp.float32)
    out_ref[...] = res[:, :-1]
    cnt = res[:, -1]

    @pl.when(jnp.max(cnt) > 1.5)
    def _fixup():
        ind = jnp.argmax(dist, axis=1)
        iota = jax.lax.broadcasted_iota(jnp.int32, dist.shape, 1)
        onehot = (iota == ind[:, None]).astype(jnp.float32)
        out_ref[...] = jnp.dot(onehot, cbn, preferred_element_type=jnp.float32)


def kernel(z, codebook):
    shape = z.shape
    d = shape[-1]
    flat = z.reshape(-1, d)
    n = flat.shape[0]
    out = pl.pallas_call(
        _vq_body,
        grid=(n // _TILE,),
        in_specs=[
            pl.BlockSpec((_TILE, d), lambda i: (i, 0)),
            pl.BlockSpec(codebook.shape, lambda i: (0, 0)),
        ],
        out_specs=pl.BlockSpec((_TILE, d), lambda i: (i, 0)),
        out_shape=jax.ShapeDtypeStruct((n, d), jnp.float32),
        compiler_params=pltpu.CompilerParams(
            dimension_semantics=("parallel",)),
    )(flat, codebook)
    return out.reshape(shape)
